# baseline (speedup 1.0000x reference)
import functools

import jax
import jax.numpy as jnp
from jax import lax
from jax.experimental import pallas as pl
from jax.experimental.pallas import tpu as pltpu
from jax.experimental.pallas import tpu_sc as plsc

MAX_DIST = 10.0
BOX = 21
NV = BOX * BOX * BOX
F = 32

NC = 2
NS = 16
NW = NC * NS
L = 16

CH = 128
CPW = 244
NSUP = 61
CPS = CPW // NSUP
NCH = NW * CPW
N_MAIN = NCH * CH
PR = N_MAIN // 4
TB = 512
NTB = N_MAIN // TB

NV_PAD = 9344
TRASH = 128
NROWS = NV_PAD + TRASH
ZROWS = NROWS // NS
OROWS = NV_PAD // NS


def _tc_transpose(x_ref, y_ref, z_ref, ft_ref, a_ref, e_ref, fp_ref, id_ref):
    tt = ft_ref[...].T
    acc = jnp.zeros((TB // 4, 128), jnp.float32)
    for q in range(4):
        acc = acc + jnp.dot(
            jnp.dot(a_ref[q], tt, preferred_element_type=jnp.float32),
            e_ref[q],
            preferred_element_type=jnp.float32,
        )
    fp_ref[...] = acc

    tx = x_ref[...] + MAX_DIST
    ty = y_ref[...] + MAX_DIST
    tz = z_ref[...] + MAX_DIST
    gx = jnp.round(tx).astype(jnp.int32)
    gy = jnp.round(ty).astype(jnp.int32)
    gz = jnp.round(tz).astype(jnp.int32)
    ok = (
        (gx >= 0) & (gx < BOX)
        & (gy >= 0) & (gy < BOX)
        & (gz >= 0) & (gz < BOX)
    )
    flat = (gx * (BOX * BOX) + gy * BOX) + gz
    pos = lax.broadcasted_iota(jnp.int32, (TB,), 0)
    trash = NV_PAD + (pos % TRASH)
    id_ref[...] = jnp.where(ok, flat, trash).reshape(1, 1, TB)


def _transpose_call(xs, ys, zs, featT):
    return pl.pallas_call(
        _tc_transpose,
        grid=(NTB,),
        in_specs=[
            pl.BlockSpec((TB,), lambda i: (i,)),
            pl.BlockSpec((TB,), lambda i: (i,)),
            pl.BlockSpec((TB,), lambda i: (i,)),
            pl.BlockSpec((F, TB), lambda i: (0, i)),
            pl.BlockSpec((4, TB // 4, TB), lambda i: (0, 0, 0)),
            pl.BlockSpec((4, F, 128), lambda i: (0, 0, 0)),
        ],
        out_specs=[
            pl.BlockSpec((TB // 4, 128), lambda i: (i, 0)),
            pl.BlockSpec((1, 1, TB), lambda i: (i, 0, 0)),
        ],
        out_shape=[
            jax.ShapeDtypeStruct((PR, 128), jnp.float32),
            jax.ShapeDtypeStruct((NTB, 1, TB), jnp.int32),
        ],
    )(xs, ys, zs, featT, _sel_a(), _sel_e())


def _sel_a():
    r = jnp.arange(TB // 4)
    p = jnp.arange(TB)
    return jnp.stack(
        [(p[None, :] == 4 * r[:, None] + q).astype(jnp.float32)
         for q in range(4)]
    )


def _sel_e():
    f = jnp.arange(F)
    j = jnp.arange(128)
    return jnp.stack(
        [(j[None, :] == q * F + f[:, None]).astype(jnp.float32)
         for q in range(4)]
    )


def _scatter_kernel():
    mesh = plsc.VectorSubcoreMesh(core_axis_name="c", subcore_axis_name="s")

    @functools.partial(
        pl.kernel,
        out_type=jax.ShapeDtypeStruct((NC, NV_PAD, F), jnp.float32),
        mesh=mesh,
        compiler_params=pltpu.CompilerParams(
            needs_layout_passes=False, use_tc_tiling_on_sc=False
        ),
        scratch_types=[
            pltpu.VMEM((CPS * CH, F), jnp.float32),
            pltpu.VMEM((CPS * CH, F), jnp.float32),
            pltpu.VMEM((CPS, CH), jnp.int32),
            pltpu.VMEM((CPS, CH), jnp.int32),
            pltpu.VMEM((ZROWS, F), jnp.float32),
            pltpu.VMEM_SHARED((NROWS, F), jnp.float32),
            pltpu.SemaphoreType.DMA,
            pltpu.SemaphoreType.DMA,
            pltpu.SemaphoreType.DMA,
            pltpu.SemaphoreType.DMA,
        ],
    )
    def k(featp, idxs, out_hbm, fb, fb2, ib, ib2, zb, acc, sf0, sf1, si0, si1):
        c = lax.axis_index("c")
        s = lax.axis_index("s")
        w = s * NC + c
        fbs = [fb, fb2]
        ibs = [ib, ib2]
        fsem = [sf0, sf1]
        isem = [si0, si1]

        zeros = jnp.zeros((L,), jnp.float32)

        def _zrow(r, carry):
            zb[r, pl.ds(0, L)] = zeros
            zb[r, pl.ds(L, L)] = zeros
            return carry
        lax.fori_loop(0, ZROWS, _zrow, None)
        pltpu.sync_copy(zb, acc.at[pl.ds(s * ZROWS, ZROWS)])
        plsc.subcore_barrier()

        base = w * CPW

        def _feat_dma(sj, b):
            ch0 = pl.multiple_of(base + sj * CPS, CPS)
            pltpu.async_copy(
                featp.at[pl.ds(ch0 * CH, CPS * CH), :], fbs[b], fsem[b]
            )
            pltpu.async_copy(idxs.at[pl.ds(ch0, CPS), :], ibs[b], isem[b])

        def _wait_dma(b):
            pltpu.make_async_copy(featp.at[pl.ds(0, CPS * CH), :], fbs[b],
                                  fsem[b]).wait()
            pltpu.make_async_copy(idxs.at[pl.ds(0, CPS), :], ibs[b],
                                  isem[b]).wait()

        _feat_dma(0, 0)
        _feat_dma(1, 1)

        def _super(sj, carry):
            for b in range(2):
                @pl.when(sj % 2 == b)
                def _():
                    _wait_dma(b)
                    for j in range(CPS):
                        pltpu.sync_copy(
                            fbs[b].at[pl.ds(j * CH, CH), :],
                            acc.at[ibs[b].at[j]],
                            add=True,
                        )

                    @pl.when(sj + 2 < NSUP)
                    def _():
                        _feat_dma(sj + 2, b)
            return carry

        lax.fori_loop(0, NSUP, _super, None)

        plsc.subcore_barrier()

        pltpu.sync_copy(acc.at[pl.ds(s * OROWS, OROWS)], zb.at[pl.ds(0, OROWS)])
        pltpu.sync_copy(
            zb.at[pl.ds(0, OROWS)], out_hbm.at[c, pl.ds(s * OROWS, OROWS), :]
        )

    return k


def _combine(p_ref, tx_ref, ty_ref, tz_ref, tf_ref, o_ref):
    g = p_ref[0, :NV, :] + p_ref[1, :NV, :]
    gx = jnp.round(tx_ref[...] + MAX_DIST).astype(jnp.int32)
    gy = jnp.round(ty_ref[...] + MAX_DIST).astype(jnp.int32)
    gz = jnp.round(tz_ref[...] + MAX_DIST).astype(jnp.int32)
    ok = (
        (gx >= 0) & (gx < BOX)
        & (gy >= 0) & (gy < BOX)
        & (gz >= 0) & (gz < BOX)
    )
    flat = jnp.where(ok, (gx * (BOX * BOX) + gy * BOX) + gz, -1)
    onehot = (
        lax.broadcasted_iota(jnp.int32, (NV, flat.shape[0]), 0) == flat[None, :]
    ).astype(jnp.float32)
    o_ref[...] = g + jnp.dot(onehot, tf_ref[...],
                             preferred_element_type=jnp.float32)


def kernel(coords, features):
    n = coords.shape[1]
    featT = jnp.swapaxes(features, 1, 2).reshape(F, n)
    xs = coords[0, :, 0]
    ys = coords[0, :, 1]
    zs = coords[0, :, 2]
    featp, idxs = _transpose_call(xs, ys, zs, featT)
    partial = _scatter_kernel()(featp.reshape(N_MAIN, F), idxs.reshape(NCH, CH))
    grid = pl.pallas_call(
        _combine,
        out_shape=jax.ShapeDtypeStruct((NV, F), jnp.float32),
    )(
        partial,
        coords[0, N_MAIN:, 0],
        coords[0, N_MAIN:, 1],
        coords[0, N_MAIN:, 2],
        features[0, N_MAIN:, :],
    )
    return grid.reshape(1, BOX, BOX, BOX, F)


# speedup vs baseline: 1.7986x; 1.7986x over previous
import functools

import jax
import jax.numpy as jnp
from jax import lax
from jax.experimental import pallas as pl
from jax.experimental.pallas import tpu as pltpu
from jax.experimental.pallas import tpu_sc as plsc

MAX_DIST = 10.0
BOX = 21
NV = BOX * BOX * BOX
F = 32

NC = 2
NS = 16
NW = NC * NS
L = 16

CH = 128
CPW = 244
NSUP = 61
CPS = CPW // NSUP
N_MAIN = NW * CPW * CH

NV_PAD = 9344
TRASH = 128
NROWS = NV_PAD + TRASH
ZROWS = NROWS // NS
OROWS = NV_PAD // NS


def _sc_scatter_kernel():
    mesh = plsc.VectorSubcoreMesh(core_axis_name="c", subcore_axis_name="s")

    @functools.partial(
        pl.kernel,
        out_type=jax.ShapeDtypeStruct((NC, NV_PAD, 128), jnp.float32),
        mesh=mesh,
        compiler_params=pltpu.CompilerParams(needs_layout_passes=False),
        scratch_types=[
            pltpu.VMEM((CPS * CH,), jnp.float32),
            pltpu.VMEM((CPS * CH,), jnp.float32),
            pltpu.VMEM((CPS * CH,), jnp.float32),
            pltpu.VMEM((2 * CPS, CH), jnp.int32),
            pltpu.VMEM((F, CH), jnp.float32),
            pltpu.VMEM((F, CH), jnp.float32),
            pltpu.VMEM((CH, 128), jnp.float32),
            pltpu.VMEM((CH, 128), jnp.float32),
            pltpu.VMEM((80, 128), jnp.float32),
            pltpu.VMEM_SHARED((NROWS, 128), jnp.float32),
            pltpu.SemaphoreType.DMA,
            pltpu.SemaphoreType.DMA,
            pltpu.SemaphoreType.DMA,
            pltpu.SemaphoreType.DMA,
        ],
    )
    def k(xs, ys, zs, feats, out_hbm, xb, yb, zb, ib, tb, tb2, pb, pb2, ob,
          acc, st0, st1, ss0, ss1):
        c = lax.axis_index("c")
        s = lax.axis_index("s")
        w = s * NC + c

        zeros = jnp.zeros((L,), jnp.float32)

        def _zrow(r, carry):
            for v in range(128 // L):
                ob[r, pl.ds(v * L, L)] = zeros
            return carry
        lax.fori_loop(0, 80, _zrow, None)

        def _prow(r, carry):
            for v in range(F // L, 128 // L):
                pb[r, pl.ds(v * L, L)] = zeros
                pb2[r, pl.ds(v * L, L)] = zeros
            return carry
        lax.fori_loop(0, CH, _prow, None)
        for kk in range(7):
            pltpu.sync_copy(ob, acc.at[pl.ds(s * ZROWS + kk * 80, 80)])
        pltpu.sync_copy(ob.at[pl.ds(0, 32)], acc.at[pl.ds(s * ZROWS + 560, 32)])

        plsc.subcore_barrier()

        lanes = lax.iota(jnp.int32, L)
        base = w * (CPW * CH)
        pbs_outer = [pb, pb2]
        ssem_outer = [ss0, ss1]

        def _stage(sj, carry):
            p0 = pl.multiple_of(base + sj * (CPS * CH), CPS * CH)
            pltpu.sync_copy(xs.at[pl.ds(p0, CPS * CH)], xb)
            pltpu.sync_copy(ys.at[pl.ds(p0, CPS * CH)], yb)
            pltpu.sync_copy(zs.at[pl.ds(p0, CPS * CH)], zb)

            parity = (sj % 2) * CPS

            def _quant(cc, carry):
                for v in range(CH // L):
                    pos = cc * CH + v * L
                    tx = xb[pl.ds(pos, L)] + MAX_DIST
                    ty = yb[pl.ds(pos, L)] + MAX_DIST
                    tz = zb[pl.ds(pos, L)] + MAX_DIST
                    ok = (
                        (tx >= -0.5) & (tx <= 20.5)
                        & (ty >= -0.5) & (ty <= 20.5)
                        & (tz >= -0.5) & (tz <= 20.5)
                    )
                    gx = jnp.minimum((tx + 0.5).astype(jnp.int32), BOX - 1)
                    gy = jnp.minimum((ty + 0.5).astype(jnp.int32), BOX - 1)
                    gz = jnp.minimum((tz + 0.5).astype(jnp.int32), BOX - 1)
                    flat = (gx * (BOX * BOX) + gy * BOX) + gz
                    trash = NV_PAD + (v * L) + lanes
                    ib[parity + cc, pl.ds(v * L, L)] = jnp.where(ok, flat, trash)
                return carry
            lax.fori_loop(0, CPS, _quant, None)

            tbs = [tb, tb2]
            pbs = [pb, pb2]
            tsem = [st0, st1]
            ssem = [ss0, ss1]

            def _mk_xpose(tbuf, pbuf):
                def _xpose(pp, carry):
                    for u in range(8):
                        p = pp * 8 + u
                        pvec = jnp.full((L,), p, jnp.int32)
                        lo = plsc.load_gather(tbuf, [lanes, pvec])
                        hi = plsc.load_gather(tbuf, [lanes + L, pvec])
                        pbuf[p, pl.ds(0, L)] = lo
                        pbuf[p, pl.ds(L, L)] = hi
                    return carry
                return _xpose

            def _tile_dma(cc, b):
                pc = pl.multiple_of(p0 + cc * CH, CH)
                return pltpu.async_copy(
                    feats.at[:, pl.ds(pc, CH)], tbs[b], tsem[b]
                )

            d = [_tile_dma(0, 0), _tile_dma(1, 1)]
            for cc in range(CPS):
                b = cc % 2
                d[b].wait()

                def _scat_wait(b=b, cc=cc):
                    pltpu.make_async_copy(
                        pbs[b], acc.at[ib.at[parity + cc]], ssem[b]
                    ).wait()

                if cc < 2:
                    pl.when(sj > 0)(_scat_wait)
                else:
                    _scat_wait()
                lax.fori_loop(0, CH // 8, _mk_xpose(tbs[b], pbs[b]), None)
                pltpu.async_copy(
                    pbs[b], acc.at[ib.at[parity + cc]], ssem[b], add=True
                )
                if cc + 2 < CPS:
                    d[b] = _tile_dma(cc + 2, b)
            return carry

        lax.fori_loop(0, NSUP, _stage, None)
        for b in range(2):
            pltpu.make_async_copy(pbs_outer[b], acc.at[ib.at[b]], ssem_outer[b]).wait()

        plsc.subcore_barrier()

        for kk in range(7):
            pltpu.sync_copy(acc.at[pl.ds(s * OROWS + kk * 80, 80)], ob)
            pltpu.sync_copy(ob, out_hbm.at[c, pl.ds(s * OROWS + kk * 80, 80), :])
        pltpu.sync_copy(acc.at[pl.ds(s * OROWS + 560, 24)], ob.at[pl.ds(0, 24)])
        pltpu.sync_copy(ob.at[pl.ds(0, 24)], out_hbm.at[c, pl.ds(s * OROWS + 560, 24), :])

    return k


def _combine(p_ref, tx_ref, ty_ref, tz_ref, tf_ref, o_ref):
    g = p_ref[0, :NV, :F] + p_ref[1, :NV, :F]
    tx = tx_ref[...] + MAX_DIST
    ty = ty_ref[...] + MAX_DIST
    tz = tz_ref[...] + MAX_DIST
    ok = (
        (tx >= -0.5) & (tx <= 20.5)
        & (ty >= -0.5) & (ty <= 20.5)
        & (tz >= -0.5) & (tz <= 20.5)
    )
    gx = jnp.minimum((tx + 0.5).astype(jnp.int32), BOX - 1)
    gy = jnp.minimum((ty + 0.5).astype(jnp.int32), BOX - 1)
    gz = jnp.minimum((tz + 0.5).astype(jnp.int32), BOX - 1)
    flat = jnp.where(ok, (gx * (BOX * BOX) + gy * BOX) + gz, -1)
    onehot = (
        lax.broadcasted_iota(jnp.int32, (NV, flat.shape[0]), 0) == flat[None, :]
    ).astype(jnp.float32)
    o_ref[...] = g + jnp.dot(onehot, tf_ref[...],
                             preferred_element_type=jnp.float32)


def kernel(coords, features):
    n = coords.shape[1]
    featT = jnp.swapaxes(features, 1, 2).reshape(F, n)
    xs = coords[0, :, 0]
    ys = coords[0, :, 1]
    zs = coords[0, :, 2]
    partial = _sc_scatter_kernel()(xs, ys, zs, featT)
    grid = pl.pallas_call(
        _combine,
        out_shape=jax.ShapeDtypeStruct((NV, F), jnp.float32),
    )(
        partial,
        coords[0, N_MAIN:, 0],
        coords[0, N_MAIN:, 1],
        coords[0, N_MAIN:, 2],
        features[0, N_MAIN:, :],
    )
    return grid.reshape(1, BOX, BOX, BOX, F)
